# baseline (speedup 1.0000x reference)
import functools

import jax
import jax.numpy as jnp
from jax import lax
from jax.experimental import pallas as pl
from jax.experimental.pallas import tpu as pltpu
from jax.experimental.pallas import tpu_sc as plsc

NC = 2
NS = 16
L = 16
NW = NC * NS
V = 1000000
D = 64
BLK = 128
NBLK = V // BLK + 1
LIM = NBLK - 1
B = 16384
PW = B // NW
SLOT = PW + L
CH = 128


def _stream_pass(t, sva, blk, tbuf, rows, lanes, tab, tail, sv2, stg,
                 semblk):
  pltpu.sync_copy(sv2.at[t], sva)
  v0 = sva[pl.ds(0, L)]
  v1 = sva[pl.ds(PW - L, L)]
  lo_b = v0[0] >> 7
  hi_end = jnp.minimum((v1[L - 1] >> 7) + 1, LIM)

  def fire(b):
    pltpu.async_copy(tab.at[:, :, pl.ds(b * BLK, BLK)], blk.at[b & 1],
                     semblk)

  def wait(b):
    pltpu.make_async_copy(tab.at[:, :, pl.ds(b * BLK, BLK)], blk.at[b & 1],
                          semblk).wait()

  def process_block(bufp, bval, ptr):
    hi_val = (bval + 1) * BLK

    def cond(st):
      return st[1]

    def body(st):
      p2, _ = st
      src = jnp.minimum(p2 + lanes, PW - 1)
      v = plsc.load_gather(sva, [src])
      m = (v < hi_val) & ((p2 + lanes) < PW)
      cnt = plsc.all_reduce_population_count(m)[0]

      @pl.when(cnt > 0)
      def _():
        c = v & (BLK - 1)
        for f in range(D):
          fb = jnp.full((L,), f // 8, jnp.int32)
          fi = jnp.full((L,), f % 8, jnp.int32)
          vals = plsc.load_gather(bufp, [fb, fi, c])
          plsc.store_scatter(rows, [src, jnp.full((L,), f, jnp.int32)],
                             vals, mask=m)

      return p2 + cnt, cnt == L

    ptr, _ = lax.while_loop(cond, body, (ptr, jnp.bool_(True)))
    return ptr

  @pl.when(lo_b < hi_end)
  def _():
    fire(lo_b)

  def bstep(b, ptr):
    wait(b)

    @pl.when(b + 1 < hi_end)
    def _():
      fire(b + 1)

    return process_block(blk.at[b & 1], b, ptr)

  ptr = lax.fori_loop(lo_b, hi_end, bstep, jnp.int32(0))

  pltpu.sync_copy(tail, tbuf)
  process_block(tbuf, jnp.int32(LIM), ptr)

  pltpu.sync_copy(rows, stg.at[pl.ds(t * PW, PW)])


def _gmf_stream_body(utab, itab, utail, itail, svu2, svi2, su_st, si_st,
                     sva, blk, tbuf, rows, semblk):
  t = lax.axis_index("s") * NC + lax.axis_index("c")
  lanes = lax.iota(jnp.int32, L)
  _stream_pass(t, sva, blk, tbuf, rows, lanes, utab, utail, svu2, su_st,
               semblk)
  _stream_pass(t, sva, blk, tbuf, rows, lanes, itab, itail, svi2, si_st,
               semblk)


def _gmf_dot_body(n_chunks, su_st, si_st, pos_u, pos_i, wb_hbm, out_hbm,
                  idx_u, idx_i, rows_u, rows_i, wb_v, out_v, sem):
  wid = lax.axis_index("s") * NC + lax.axis_index("c")

  pltpu.sync_copy(pos_u.at[wid], idx_u)
  pltpu.sync_copy(pos_i.at[wid], idx_i)
  pltpu.sync_copy(wb_hbm, wb_v)

  copies = []
  for j in range(n_chunks):
    copies.append(
        pltpu.async_copy(su_st.at[idx_u.at[j]], rows_u.at[pl.ds(j * CH, CH)],
                         sem))
    copies.append(
        pltpu.async_copy(si_st.at[idx_i.at[j]], rows_i.at[pl.ds(j * CH, CH)],
                         sem))
  for c in copies:
    c.wait()

  lanes = lax.iota(jnp.int32, L)
  bias = wb_v[pl.ds(D * L, L)]

  def group(g, _):
    rid = g * L + lanes

    def feat(f, acc):
      col = jnp.full((L,), f, jnp.int32)
      uvec = plsc.load_gather(rows_u, [rid, col])
      ivec = plsc.load_gather(rows_i, [rid, col])
      wvec = wb_v[pl.ds(f * L, L)]
      return acc + uvec * ivec * wvec

    acc = lax.fori_loop(0, D, feat, jnp.zeros((L,), jnp.float32))
    out_v[pl.ds(g * L, L)] = 1.0 / (1.0 + jnp.exp(-(acc + bias)))
    return 0

  lax.fori_loop(0, PW // L, group, 0)
  pltpu.sync_copy(out_v, out_hbm.at[pl.ds(wid * PW, PW)])


def kernel(user_indices, item_indices, user_table, item_table, W, b):
  utab3 = user_table.T.reshape(8, 8, V)
  itab3 = item_table.T.reshape(8, 8, V)
  pad = ((0, 0), (0, BLK - (V - LIM * BLK)))
  utail = jnp.pad(user_table[LIM * BLK:].T, pad).reshape(8, 8, BLK)
  itail = jnp.pad(item_table[LIM * BLK:].T, pad).reshape(8, 8, BLK)

  ui = user_indices.astype(jnp.int32)
  ii = item_indices.astype(jnp.int32)
  osu = jnp.argsort(ui)
  osi = jnp.argsort(ii)
  svu2 = ui[osu].reshape(NW, PW)
  svi2 = ii[osi].reshape(NW, PW)
  ar = jnp.arange(B, dtype=jnp.int32)
  pu = jnp.zeros((B,), jnp.int32).at[osu].set(ar)
  pi = jnp.zeros((B,), jnp.int32).at[osi].set(ar)
  pos_u = pu.reshape(NW, PW // CH, CH)
  pos_i = pi.reshape(NW, PW // CH, CH)

  wb = jnp.repeat(jnp.concatenate([W[:, 0], b]), L).astype(jnp.float32)

  mesh = plsc.VectorSubcoreMesh(
      core_axis_name="c", subcore_axis_name="s", num_cores=NC, num_subcores=NS)
  params = pltpu.CompilerParams(needs_layout_passes=False)

  stream = pl.kernel(
      _gmf_stream_body,
      out_type=(
          jax.ShapeDtypeStruct((B, D), jnp.float32),
          jax.ShapeDtypeStruct((B, D), jnp.float32),
      ),
      mesh=mesh,
      compiler_params=params,
      scratch_types=[
          pltpu.VMEM((PW,), jnp.int32),
          pltpu.VMEM((2, 8, 8, BLK), jnp.float32),
          pltpu.VMEM((8, 8, BLK), jnp.float32),
          pltpu.VMEM((PW, D), jnp.float32),
          pltpu.SemaphoreType.DMA,
      ],
  )
  su_st, si_st = stream(utab3, itab3, utail, itail, svu2, svi2)

  dot = pl.kernel(
      functools.partial(_gmf_dot_body, PW // CH),
      out_type=jax.ShapeDtypeStruct((B,), jnp.float32),
      mesh=mesh,
      compiler_params=pltpu.CompilerParams(
          needs_layout_passes=False, use_tc_tiling_on_sc=False),
      scratch_types=[
          pltpu.VMEM((PW // CH, CH), jnp.int32),
          pltpu.VMEM((PW // CH, CH), jnp.int32),
          pltpu.VMEM((PW, D), jnp.float32),
          pltpu.VMEM((PW, D), jnp.float32),
          pltpu.VMEM(((D + 1) * L,), jnp.float32),
          pltpu.VMEM((PW,), jnp.float32),
          pltpu.SemaphoreType.DMA,
      ],
  )
  return dot(su_st, si_st, pos_u, pos_i, wb)


# speedup vs baseline: 1.3350x; 1.3350x over previous
import functools

import jax
import jax.numpy as jnp
from jax import lax
from jax.experimental import pallas as pl
from jax.experimental.pallas import tpu as pltpu
from jax.experimental.pallas import tpu_sc as plsc

NC = 2
NS = 16
L = 16
NW = NC * NS
V = 1000000
D = 64
BLK = 128
SW = 256
LIM = V // SW
B = 16384
PW = B // NW
SLOT = PW + L
CH = 128


def _stream_pass(t, sva, blk, tbuf, rows, lanes, tab, tail, sv2, stg,
                 semblk):
  pltpu.sync_copy(sv2.at[t], sva)
  v0 = sva[pl.ds(0, L)]
  v1 = sva[pl.ds(PW - L, L)]
  lo_b = v0[0] // SW
  hi_end = jnp.minimum(v1[L - 1] // SW + 1, LIM)

  def fire(b):
    pltpu.async_copy(tab.at[:, :, pl.ds(b * SW, SW)], blk.at[b & 1],
                     semblk)

  def wait(b):
    pltpu.make_async_copy(tab.at[:, :, pl.ds(b * SW, SW)], blk.at[b & 1],
                          semblk).wait()

  def process_block(bufp, bval, ptr):
    hi_val = (bval + 1) * SW

    def cond(st):
      return st[1]

    def body(st):
      p2, _ = st
      src = jnp.minimum(p2 + lanes, PW - 1)
      v = plsc.load_gather(sva, [src])
      m = (v < hi_val) & ((p2 + lanes) < PW)
      cnt = plsc.all_reduce_population_count(m)[0]

      @pl.when(cnt > 0)
      def _():
        c = v - bval * SW
        for f in range(D):
          fb = jnp.full((L,), f // 8, jnp.int32)
          fi = jnp.full((L,), f % 8, jnp.int32)
          vals = plsc.load_gather(bufp, [fb, fi, c])
          plsc.store_scatter(rows, [src, jnp.full((L,), f, jnp.int32)],
                             vals, mask=m)

      return p2 + cnt, cnt == L

    ptr, _ = lax.while_loop(cond, body, (ptr, jnp.bool_(True)))
    return ptr

  @pl.when(lo_b < hi_end)
  def _():
    fire(lo_b)

  def bstep(b, ptr):
    wait(b)

    @pl.when(b + 1 < hi_end)
    def _():
      fire(b + 1)

    return process_block(blk.at[b & 1], b, ptr)

  ptr = lax.fori_loop(lo_b, hi_end, bstep, jnp.int32(0))

  pltpu.sync_copy(tail, tbuf)
  process_block(tbuf, jnp.int32(LIM), ptr)

  pltpu.sync_copy(rows, stg.at[pl.ds(t * PW, PW)])


def _gmf_stream_body(utab, itab, utail, itail, svu2, svi2, su_st, si_st,
                     sva, blk, tbuf, rows, semblk):
  t = lax.axis_index("s") * NC + lax.axis_index("c")
  lanes = lax.iota(jnp.int32, L)
  _stream_pass(t, sva, blk, tbuf, rows, lanes, utab, utail, svu2, su_st,
               semblk)
  _stream_pass(t, sva, blk, tbuf, rows, lanes, itab, itail, svi2, si_st,
               semblk)


def _gmf_dot_body(n_chunks, su_st, si_st, pos_u, pos_i, wb_hbm, out_hbm,
                  idx_u, idx_i, rows_u, rows_i, wb_v, out_v, sem):
  wid = lax.axis_index("s") * NC + lax.axis_index("c")

  pltpu.sync_copy(pos_u.at[wid], idx_u)
  pltpu.sync_copy(pos_i.at[wid], idx_i)
  pltpu.sync_copy(wb_hbm, wb_v)

  copies = []
  for j in range(n_chunks):
    copies.append(
        pltpu.async_copy(su_st.at[idx_u.at[j]], rows_u.at[pl.ds(j * CH, CH)],
                         sem))
    copies.append(
        pltpu.async_copy(si_st.at[idx_i.at[j]], rows_i.at[pl.ds(j * CH, CH)],
                         sem))
  for c in copies:
    c.wait()

  lanes = lax.iota(jnp.int32, L)
  bias = wb_v[pl.ds(D * L, L)]

  def group(g, _):
    rid = g * L + lanes

    def feat(f, acc):
      col = jnp.full((L,), f, jnp.int32)
      uvec = plsc.load_gather(rows_u, [rid, col])
      ivec = plsc.load_gather(rows_i, [rid, col])
      wvec = wb_v[pl.ds(f * L, L)]
      return acc + uvec * ivec * wvec

    acc = lax.fori_loop(0, D, feat, jnp.zeros((L,), jnp.float32))
    out_v[pl.ds(g * L, L)] = 1.0 / (1.0 + jnp.exp(-(acc + bias)))
    return 0

  lax.fori_loop(0, PW // L, group, 0)
  pltpu.sync_copy(out_v, out_hbm.at[pl.ds(wid * PW, PW)])


def kernel(user_indices, item_indices, user_table, item_table, W, b):
  utab3 = user_table.T.reshape(8, 8, V)
  itab3 = item_table.T.reshape(8, 8, V)
  pad = ((0, 0), (0, SW - (V - LIM * SW)))
  utail = jnp.pad(user_table[LIM * SW:].T, pad).reshape(8, 8, SW)
  itail = jnp.pad(item_table[LIM * SW:].T, pad).reshape(8, 8, SW)

  ui = user_indices.astype(jnp.int32)
  ii = item_indices.astype(jnp.int32)
  osu = jnp.argsort(ui)
  osi = jnp.argsort(ii)
  svu2 = ui[osu].reshape(NW, PW)
  svi2 = ii[osi].reshape(NW, PW)
  ar = jnp.arange(B, dtype=jnp.int32)
  pu = jnp.zeros((B,), jnp.int32).at[osu].set(ar)
  pi = jnp.zeros((B,), jnp.int32).at[osi].set(ar)
  pos_u = pu.reshape(NW, PW // CH, CH)
  pos_i = pi.reshape(NW, PW // CH, CH)

  wb = jnp.repeat(jnp.concatenate([W[:, 0], b]), L).astype(jnp.float32)

  mesh = plsc.VectorSubcoreMesh(
      core_axis_name="c", subcore_axis_name="s", num_cores=NC, num_subcores=NS)
  params = pltpu.CompilerParams(needs_layout_passes=False)

  stream = pl.kernel(
      _gmf_stream_body,
      out_type=(
          jax.ShapeDtypeStruct((B, D), jnp.float32),
          jax.ShapeDtypeStruct((B, D), jnp.float32),
      ),
      mesh=mesh,
      compiler_params=params,
      scratch_types=[
          pltpu.VMEM((PW,), jnp.int32),
          pltpu.VMEM((2, 8, 8, SW), jnp.float32),
          pltpu.VMEM((8, 8, SW), jnp.float32),
          pltpu.VMEM((PW, D), jnp.float32),
          pltpu.SemaphoreType.DMA,
      ],
  )
  su_st, si_st = stream(utab3, itab3, utail, itail, svu2, svi2)

  dot = pl.kernel(
      functools.partial(_gmf_dot_body, PW // CH),
      out_type=jax.ShapeDtypeStruct((B,), jnp.float32),
      mesh=mesh,
      compiler_params=pltpu.CompilerParams(
          needs_layout_passes=False, use_tc_tiling_on_sc=False),
      scratch_types=[
          pltpu.VMEM((PW // CH, CH), jnp.int32),
          pltpu.VMEM((PW // CH, CH), jnp.int32),
          pltpu.VMEM((PW, D), jnp.float32),
          pltpu.VMEM((PW, D), jnp.float32),
          pltpu.VMEM(((D + 1) * L,), jnp.float32),
          pltpu.VMEM((PW,), jnp.float32),
          pltpu.SemaphoreType.DMA,
      ],
  )
  return dot(su_st, si_st, pos_u, pos_i, wb)


# speedup vs baseline: 1.5265x; 1.1434x over previous
import functools

import jax
import jax.numpy as jnp
from jax import lax
from jax.experimental import pallas as pl
from jax.experimental.pallas import tpu as pltpu
from jax.experimental.pallas import tpu_sc as plsc

NC = 2
NS = 16
L = 16
NW = NC * NS
V = 1000000
D = 64
BLK = 128
SW = 384
TW = 128
LIM = V // SW
B = 16384
PW = B // NW
SLOT = PW + L
CH = 128


def _stream_pass(t, sva, blk, tbuf, rows, lanes, tab, tail, sv2, stg,
                 semblk):
  pltpu.sync_copy(sv2.at[t], sva)
  v0 = sva[pl.ds(0, L)]
  v1 = sva[pl.ds(PW - L, L)]
  lo_b = v0[0] // SW
  hi_end = jnp.minimum(v1[L - 1] // SW + 1, LIM)

  def fire(b):
    pltpu.async_copy(tab.at[:, :, pl.ds(b * SW, SW)], blk.at[b & 1],
                     semblk)

  def wait(b):
    pltpu.make_async_copy(tab.at[:, :, pl.ds(b * SW, SW)], blk.at[b & 1],
                          semblk).wait()

  def process_block(bufp, bval, ptr):
    hi_val = (bval + 1) * SW

    def cond(st):
      return st[1]

    def body(st):
      p2, _ = st
      src = jnp.minimum(p2 + lanes, PW - 1)
      v = plsc.load_gather(sva, [src])
      m = (v < hi_val) & ((p2 + lanes) < PW)
      cnt = plsc.all_reduce_population_count(m)[0]

      @pl.when(cnt > 0)
      def _():
        c = v - bval * SW
        for f in range(D):
          fb = jnp.full((L,), f // 8, jnp.int32)
          fi = jnp.full((L,), f % 8, jnp.int32)
          vals = plsc.load_gather(bufp, [fb, fi, c])
          plsc.store_scatter(rows, [src, jnp.full((L,), f, jnp.int32)],
                             vals, mask=m)

      return p2 + cnt, cnt == L

    ptr, _ = lax.while_loop(cond, body, (ptr, jnp.bool_(True)))
    return ptr

  @pl.when(lo_b < hi_end)
  def _():
    fire(lo_b)

  def bstep(b, ptr):
    wait(b)

    @pl.when(b + 1 < hi_end)
    def _():
      fire(b + 1)

    return process_block(blk.at[b & 1], b, ptr)

  ptr = lax.fori_loop(lo_b, hi_end, bstep, jnp.int32(0))

  pltpu.sync_copy(tail, tbuf)
  process_block(tbuf, jnp.int32(LIM), ptr)

  pltpu.sync_copy(rows, stg.at[pl.ds(t * PW, PW)])


def _gmf_stream_body(utab, itab, utail, itail, svu2, svi2, su_st, si_st,
                     sva, blk, tbuf, rows, semblk):
  t = lax.axis_index("s") * NC + lax.axis_index("c")
  lanes = lax.iota(jnp.int32, L)
  _stream_pass(t, sva, blk, tbuf, rows, lanes, utab, utail, svu2, su_st,
               semblk)
  _stream_pass(t, sva, blk, tbuf, rows, lanes, itab, itail, svi2, si_st,
               semblk)


def _gmf_dot_body(n_chunks, su_st, si_st, pos_u, pos_i, wb_hbm, out_hbm,
                  idx_u, idx_i, rows_u, rows_i, wb_v, out_v, sem):
  wid = lax.axis_index("s") * NC + lax.axis_index("c")

  pltpu.sync_copy(pos_u.at[wid], idx_u)
  pltpu.sync_copy(pos_i.at[wid], idx_i)
  pltpu.sync_copy(wb_hbm, wb_v)

  copies = []
  for j in range(n_chunks):
    copies.append(
        pltpu.async_copy(su_st.at[idx_u.at[j]], rows_u.at[pl.ds(j * CH, CH)],
                         sem))
    copies.append(
        pltpu.async_copy(si_st.at[idx_i.at[j]], rows_i.at[pl.ds(j * CH, CH)],
                         sem))
  for c in copies:
    c.wait()

  lanes = lax.iota(jnp.int32, L)
  bias = wb_v[pl.ds(D * L, L)]

  def group(g, _):
    rid = g * L + lanes

    def feat(f, acc):
      col = jnp.full((L,), f, jnp.int32)
      uvec = plsc.load_gather(rows_u, [rid, col])
      ivec = plsc.load_gather(rows_i, [rid, col])
      wvec = wb_v[pl.ds(f * L, L)]
      return acc + uvec * ivec * wvec

    acc = lax.fori_loop(0, D, feat, jnp.zeros((L,), jnp.float32))
    out_v[pl.ds(g * L, L)] = 1.0 / (1.0 + jnp.exp(-(acc + bias)))
    return 0

  lax.fori_loop(0, PW // L, group, 0)
  pltpu.sync_copy(out_v, out_hbm.at[pl.ds(wid * PW, PW)])


def kernel(user_indices, item_indices, user_table, item_table, W, b):
  utab3 = user_table.T.reshape(8, 8, V)
  itab3 = item_table.T.reshape(8, 8, V)
  pad = ((0, 0), (0, TW - (V - LIM * SW)))
  utail = jnp.pad(user_table[LIM * SW:].T, pad).reshape(8, 8, TW)
  itail = jnp.pad(item_table[LIM * SW:].T, pad).reshape(8, 8, TW)

  ui = user_indices.astype(jnp.int32)
  ii = item_indices.astype(jnp.int32)
  osu = jnp.argsort(ui)
  osi = jnp.argsort(ii)
  svu2 = ui[osu].reshape(NW, PW)
  svi2 = ii[osi].reshape(NW, PW)
  ar = jnp.arange(B, dtype=jnp.int32)
  pu = jnp.zeros((B,), jnp.int32).at[osu].set(ar)
  pi = jnp.zeros((B,), jnp.int32).at[osi].set(ar)
  pos_u = pu.reshape(NW, PW // CH, CH)
  pos_i = pi.reshape(NW, PW // CH, CH)

  wb = jnp.repeat(jnp.concatenate([W[:, 0], b]), L).astype(jnp.float32)

  mesh = plsc.VectorSubcoreMesh(
      core_axis_name="c", subcore_axis_name="s", num_cores=NC, num_subcores=NS)
  params = pltpu.CompilerParams(needs_layout_passes=False)

  stream = pl.kernel(
      _gmf_stream_body,
      out_type=(
          jax.ShapeDtypeStruct((B, D), jnp.float32),
          jax.ShapeDtypeStruct((B, D), jnp.float32),
      ),
      mesh=mesh,
      compiler_params=params,
      scratch_types=[
          pltpu.VMEM((PW,), jnp.int32),
          pltpu.VMEM((2, 8, 8, SW), jnp.float32),
          pltpu.VMEM((8, 8, TW), jnp.float32),
          pltpu.VMEM((PW, D), jnp.float32),
          pltpu.SemaphoreType.DMA,
      ],
  )
  su_st, si_st = stream(utab3, itab3, utail, itail, svu2, svi2)

  dot = pl.kernel(
      functools.partial(_gmf_dot_body, PW // CH),
      out_type=jax.ShapeDtypeStruct((B,), jnp.float32),
      mesh=mesh,
      compiler_params=pltpu.CompilerParams(
          needs_layout_passes=False, use_tc_tiling_on_sc=False),
      scratch_types=[
          pltpu.VMEM((PW // CH, CH), jnp.int32),
          pltpu.VMEM((PW // CH, CH), jnp.int32),
          pltpu.VMEM((PW, D), jnp.float32),
          pltpu.VMEM((PW, D), jnp.float32),
          pltpu.VMEM(((D + 1) * L,), jnp.float32),
          pltpu.VMEM((PW,), jnp.float32),
          pltpu.SemaphoreType.DMA,
      ],
  )
  return dot(su_st, si_st, pos_u, pos_i, wb)
